# baseline (speedup 1.0000x reference)
import jax
import jax.numpy as jnp
from jax import lax
from jax.experimental import pallas as pl
from jax.experimental.pallas import tpu as pltpu
from jax.experimental.pallas import tpu_sc as plsc

_N = 10000
_T = 12
_F = 128
_FO = 128
_E = 320000

_NCORE = 2
_NSUB = 16
_CH = 80
_EPT = _E // _NSUB
_NCHUNK = _EPT // _CH
_RPT = _N // _NSUB
_DW = 16
_TPC = _T // _NCORE

_mesh = plsc.VectorSubcoreMesh(
    core_axis_name="c", subcore_axis_name="s", num_cores=_NCORE,
    num_subcores=_NSUB)
_sc_params = pltpu.CompilerParams(use_tc_tiling_on_sc=False)


def _deg_body(dst_hbm, zeros_hbm, ones_hbm, deg_hbm, dst_v, ones_v, deg_sh):
    c = lax.axis_index("c")
    s = lax.axis_index("s")

    @pl.when(c == 0)
    def _():
        pltpu.sync_copy(dst_hbm.at[s], dst_v)
        pltpu.sync_copy(ones_hbm, ones_v)

        @pl.when(s == 0)
        def _():
            pltpu.sync_copy(zeros_hbm, deg_sh)

        plsc.subcore_barrier()

        def chunk(j, carry):
            pltpu.sync_copy(ones_v, deg_sh.at[dst_v.at[j]], add=True)
            return carry

        lax.fori_loop(0, _NCHUNK, chunk, 0)
        plsc.subcore_barrier()
        pltpu.sync_copy(deg_sh.at[pl.ds(s * _RPT, _RPT)],
                        deg_hbm.at[pl.ds(s * _RPT, _RPT)])


_deg_call = pl.kernel(
    _deg_body,
    out_type=jax.ShapeDtypeStruct((_N, _DW), jnp.float32),
    mesh=_mesh,
    compiler_params=_sc_params,
    scratch_types=[
        pltpu.VMEM((_NCHUNK, _CH), jnp.int32),
        pltpu.VMEM((_CH, _DW), jnp.float32),
        pltpu.VMEM_SHARED((_N, _DW), jnp.float32),
    ],
)


def _agg_body(xn_hbm, src_hbm, dst_hbm, s_hbm, src_v,
              rows0, rows1, dstc0, dstc1, s_sh,
              semr0, semr1, semd0, semd1, semsc0, semsc1):
    c = lax.axis_index("c")
    s = lax.axis_index("s")
    pltpu.sync_copy(src_hbm.at[s], src_v)
    bufs = ((rows0, dstc0, semr0, semd0, semsc0),
            (rows1, dstc1, semr1, semd1, semsc1))

    for i in range(_TPC):
        t = c * _TPC + i
        toff = t * _N

        step = lax.select(i == 0, c * _TPC * _N, _N)

        def offrow(j, carry):
            for k in range(_CH // 16):
                sl = pl.ds(k * 16, 16)
                src_v[j, sl] = src_v[j, sl] + step
            return carry

        lax.fori_loop(0, _NCHUNK, offrow, 0)

        pltpu.async_copy(xn_hbm.at[src_v.at[0]], rows0, semr0)
        pltpu.async_copy(dst_hbm.at[s, 0], dstc0.at[0], semd0)

        pltpu.sync_copy(xn_hbm.at[pl.ds(toff + s * _RPT, _RPT)],
                        s_sh.at[pl.ds(s * _RPT, _RPT)])
        plsc.subcore_barrier()

        def chunk2(jj, carry):
            for b, (rows, dstc, semr, semd, semsc) in enumerate(bufs):
                j = 2 * jj + b
                nrows, ndstc, nsemr, nsemd, nsemsc = bufs[1 - b]

                @pl.when(j < _NCHUNK - 1)
                def _():
                    @pl.when(j >= 1)
                    def _():
                        pltpu.make_async_copy(nrows, s_sh.at[ndstc.at[0]],
                                              nsemsc).wait()

                    pltpu.async_copy(xn_hbm.at[src_v.at[j + 1]], nrows, nsemr)
                    pltpu.async_copy(dst_hbm.at[s, j + 1], ndstc.at[0], nsemd)

                pltpu.make_async_copy(xn_hbm.at[src_v.at[j]], rows,
                                      semr).wait()
                pltpu.make_async_copy(dst_hbm.at[s, j], dstc.at[0],
                                      semd).wait()
                pltpu.async_copy(rows, s_sh.at[dstc.at[0]], semsc, add=True)
            return carry

        lax.fori_loop(0, _NCHUNK // 2, chunk2, 0)
        pltpu.make_async_copy(rows0, s_sh.at[dstc0.at[0]], semsc0).wait()
        pltpu.make_async_copy(rows1, s_sh.at[dstc1.at[0]], semsc1).wait()
        plsc.subcore_barrier()
        pltpu.sync_copy(s_sh.at[pl.ds(s * _RPT, _RPT)],
                        s_hbm.at[pl.ds(toff + s * _RPT, _RPT)])
        plsc.subcore_barrier()


_agg_call = pl.kernel(
    _agg_body,
    out_type=jax.ShapeDtypeStruct((_T * _N, _F), jnp.float32),
    mesh=_mesh,
    compiler_params=_sc_params,
    scratch_types=[
        pltpu.VMEM((_NCHUNK, _CH), jnp.int32),
        pltpu.VMEM((_CH, _F), jnp.float32),
        pltpu.VMEM((_CH, _F), jnp.float32),
        pltpu.VMEM((1, _CH), jnp.int32),
        pltpu.VMEM((1, _CH), jnp.int32),
        pltpu.VMEM_SHARED((_N, _F), jnp.float32),
        pltpu.SemaphoreType.DMA,
        pltpu.SemaphoreType.DMA,
        pltpu.SemaphoreType.DMA,
        pltpu.SemaphoreType.DMA,
        pltpu.SemaphoreType.DMA,
        pltpu.SemaphoreType.DMA,
    ],
)


def _prescale_body(x_ref, deg_ref, out_ref):
    rn = lax.rsqrt(deg_ref[:, 0:1] + 1.0)
    out_ref[...] = x_ref[0] * rn


_NB = 10
_BN = _N // _NB


@jax.jit
def _prescale(X, deg_w):
    return pl.pallas_call(
        _prescale_body,
        grid=(_T, _NB),
        in_specs=[
            pl.BlockSpec((1, _BN, _F), lambda t, n: (t, n, 0)),
            pl.BlockSpec((_BN, _DW), lambda t, n: (n, 0)),
        ],
        out_specs=pl.BlockSpec((_BN, _F), lambda t, n: (t * _NB + n, 0)),
        out_shape=jax.ShapeDtypeStruct((_T * _N, _F), jnp.float32),
    )(X, deg_w)


def _final_body(s_ref, deg_ref, w_ref, b_ref, out_ref):
    rn = lax.rsqrt(deg_ref[:, 0:1] + 1.0)
    sn = s_ref[...] * rn
    p = jnp.dot(sn, w_ref[...], preferred_element_type=jnp.float32) + b_ref[...]
    r = jax.nn.sigmoid(p[:, :_FO])
    hc = jnp.tanh(p[:, _FO:])
    out_ref[...] = (1.0 - r) * hc


@jax.jit
def _final(S, deg_w, Wc, bc):
    return pl.pallas_call(
        _final_body,
        grid=(_T * _NB,),
        in_specs=[
            pl.BlockSpec((_BN, _F), lambda m: (m, 0)),
            pl.BlockSpec((_BN, _DW), lambda m: (m % _NB, 0)),
            pl.BlockSpec((_F, 2 * _FO), lambda m: (0, 0)),
            pl.BlockSpec((1, 2 * _FO), lambda m: (0, 0)),
        ],
        out_specs=pl.BlockSpec((_BN, _FO), lambda m: (m, 0)),
        out_shape=jax.ShapeDtypeStruct((_T * _N, _FO), jnp.float32),
    )(S, deg_w, Wc, bc)


def kernel(X, edge_index, W_gate, b_gate, W_upd, b_upd):
    src = edge_index[0].reshape(_NSUB, _NCHUNK, _CH)
    dst = edge_index[1].reshape(_NSUB, _NCHUNK, _CH)
    zeros = jnp.zeros((_N, _DW), jnp.float32)
    ones = jnp.ones((_CH, _DW), jnp.float32)
    deg_w = _deg_call(dst, zeros, ones)
    Xn = _prescale(X, deg_w)
    S = _agg_call(Xn, src, dst)
    Wc = jnp.concatenate([W_gate[:_F, _FO:], W_upd[:_F]], axis=1)
    bc = jnp.concatenate([b_gate[_FO:], b_upd]).reshape(1, 2 * _FO)
    H = _final(S, deg_w, Wc, bc)
    return H.reshape(_T, _N, _FO)


# speedup vs baseline: 1.1321x; 1.1321x over previous
import jax
import jax.numpy as jnp
from jax import lax
from jax.experimental import pallas as pl
from jax.experimental.pallas import tpu as pltpu
from jax.experimental.pallas import tpu_sc as plsc

_N = 10000
_T = 12
_F = 128
_FO = 128
_E = 320000

_NCORE = 2
_NSUB = 16
_CH = 80
_EPT = _E // _NSUB
_NCHUNK = _EPT // _CH
_ACH = 125
_ANCH = _EPT // _ACH
_NBUF = 3
_RPT = _N // _NSUB
_DW = 16
_TPC = _T // _NCORE

_mesh = plsc.VectorSubcoreMesh(
    core_axis_name="c", subcore_axis_name="s", num_cores=_NCORE,
    num_subcores=_NSUB)
_sc_params = pltpu.CompilerParams(use_tc_tiling_on_sc=False)


def _deg_body(dst_hbm, zeros_hbm, ones_hbm, deg_hbm, dst_v, ones_v, deg_sh):
    c = lax.axis_index("c")
    s = lax.axis_index("s")

    @pl.when(c == 0)
    def _():
        pltpu.sync_copy(dst_hbm.at[s], dst_v)
        pltpu.sync_copy(ones_hbm, ones_v)

        @pl.when(s == 0)
        def _():
            pltpu.sync_copy(zeros_hbm, deg_sh)

        plsc.subcore_barrier()

        def chunk(j, carry):
            pltpu.sync_copy(ones_v, deg_sh.at[dst_v.at[j]], add=True)
            return carry

        lax.fori_loop(0, _NCHUNK, chunk, 0)
        plsc.subcore_barrier()
        pltpu.sync_copy(deg_sh.at[pl.ds(s * _RPT, _RPT)],
                        deg_hbm.at[pl.ds(s * _RPT, _RPT)])


_deg_call = pl.kernel(
    _deg_body,
    out_type=jax.ShapeDtypeStruct((_N, _DW), jnp.float32),
    mesh=_mesh,
    compiler_params=_sc_params,
    scratch_types=[
        pltpu.VMEM((_NCHUNK, _CH), jnp.int32),
        pltpu.VMEM((_CH, _DW), jnp.float32),
        pltpu.VMEM_SHARED((_N, _DW), jnp.float32),
    ],
)


def _agg_body(xn_hbm, comb_hbm, s_hbm,
              rows0, rows1, rows2, idx0, idx1, idx2, s_sh,
              semr0, semr1, semr2, semi0, semi1, semi2,
              semsc0, semsc1, semsc2, semw):
    c = lax.axis_index("c")
    s = lax.axis_index("s")
    bufs = ((rows0, idx0, semr0, semi0, semsc0),
            (rows1, idx1, semr1, semi1, semsc1),
            (rows2, idx2, semr2, semi2, semsc2))

    def fire_idx(j, buf):
        pltpu.async_copy(comb_hbm.at[s, j], buf[1], buf[3])

    def fire_gather(tbl, j, buf):
        pltpu.make_async_copy(comb_hbm.at[s, j], buf[1], buf[3]).wait()
        pltpu.async_copy(tbl.at[buf[1].at[0]], buf[0], buf[2])

    def wait_gather_fire_scatter(tbl, buf):
        pltpu.make_async_copy(tbl.at[buf[1].at[0]], buf[0], buf[2]).wait()
        pltpu.async_copy(buf[0], s_sh.at[buf[1].at[1]], buf[4], add=True)

    def drain_scatter(buf):
        pltpu.make_async_copy(buf[0], s_sh.at[buf[1].at[1]], buf[4]).wait()

    for i in range(_TPC):
        t = c * _TPC + i
        toff = t * _N
        tbl = xn_hbm.at[pl.ds(toff, _N)]

        fire_idx(0, bufs[0])
        fire_idx(1, bufs[1])
        fire_gather(tbl, 0, bufs[0])

        @pl.when(i > 0)
        def _():
            pltpu.make_async_copy(
                s_sh.at[pl.ds(s * _RPT, _RPT)],
                s_hbm.at[pl.ds((toff - _N) + s * _RPT, _RPT)], semw).wait()

        pltpu.sync_copy(xn_hbm.at[pl.ds(toff + s * _RPT, _RPT)],
                        s_sh.at[pl.ds(s * _RPT, _RPT)])
        plsc.subcore_barrier()

        def chunk3(jj, carry):
            for b in range(_NBUF):
                j = _NBUF * jj + b

                @pl.when(j >= 1)
                def _():
                    drain_scatter(bufs[(b + _NBUF - 1) % _NBUF])

                @pl.when(j + 2 <= _ANCH - 1)
                def _():
                    fire_idx(j + 2, bufs[(b + 2) % _NBUF])

                @pl.when(j + 1 <= _ANCH - 1)
                def _():
                    fire_gather(tbl, j + 1, bufs[(b + 1) % _NBUF])

                wait_gather_fire_scatter(tbl, bufs[b])
            return carry

        lax.fori_loop(0, _ANCH // _NBUF, chunk3, 0)
        drain_scatter(bufs[(_ANCH - 2) % _NBUF])
        wait_gather_fire_scatter(tbl, bufs[(_ANCH - 1) % _NBUF])
        drain_scatter(bufs[(_ANCH - 1) % _NBUF])
        plsc.subcore_barrier()
        pltpu.async_copy(s_sh.at[pl.ds(s * _RPT, _RPT)],
                         s_hbm.at[pl.ds(toff + s * _RPT, _RPT)], semw)

    pltpu.make_async_copy(
        s_sh.at[pl.ds(s * _RPT, _RPT)],
        s_hbm.at[pl.ds((c * _TPC + _TPC - 1) * _N + s * _RPT, _RPT)],
        semw).wait()


_agg_call = pl.kernel(
    _agg_body,
    out_type=jax.ShapeDtypeStruct((_T * _N, _F), jnp.float32),
    mesh=_mesh,
    compiler_params=_sc_params,
    scratch_types=[
        pltpu.VMEM((_ACH, _F), jnp.float32),
        pltpu.VMEM((_ACH, _F), jnp.float32),
        pltpu.VMEM((_ACH, _F), jnp.float32),
        pltpu.VMEM((2, _ACH), jnp.int32),
        pltpu.VMEM((2, _ACH), jnp.int32),
        pltpu.VMEM((2, _ACH), jnp.int32),
        pltpu.VMEM_SHARED((_N, _F), jnp.float32),
        pltpu.SemaphoreType.DMA,
        pltpu.SemaphoreType.DMA,
        pltpu.SemaphoreType.DMA,
        pltpu.SemaphoreType.DMA,
        pltpu.SemaphoreType.DMA,
        pltpu.SemaphoreType.DMA,
        pltpu.SemaphoreType.DMA,
        pltpu.SemaphoreType.DMA,
        pltpu.SemaphoreType.DMA,
        pltpu.SemaphoreType.DMA,
    ],
)


def _prescale_body(x_ref, deg_ref, out_ref):
    rn = lax.rsqrt(deg_ref[:, 0:1] + 1.0)
    out_ref[...] = x_ref[0] * rn


_NB = 10
_BN = _N // _NB


@jax.jit
def _prescale(X, deg_w):
    return pl.pallas_call(
        _prescale_body,
        grid=(_T, _NB),
        in_specs=[
            pl.BlockSpec((1, _BN, _F), lambda t, n: (t, n, 0)),
            pl.BlockSpec((_BN, _DW), lambda t, n: (n, 0)),
        ],
        out_specs=pl.BlockSpec((_BN, _F), lambda t, n: (t * _NB + n, 0)),
        out_shape=jax.ShapeDtypeStruct((_T * _N, _F), jnp.float32),
    )(X, deg_w)


def _final_body(s_ref, deg_ref, w_ref, b_ref, out_ref):
    rn = lax.rsqrt(deg_ref[:, 0:1] + 1.0)
    sn = s_ref[...] * rn
    p = jnp.dot(sn, w_ref[...], preferred_element_type=jnp.float32) + b_ref[...]
    r = jax.nn.sigmoid(p[:, :_FO])
    hc = jnp.tanh(p[:, _FO:])
    out_ref[...] = (1.0 - r) * hc


@jax.jit
def _final(S, deg_w, Wc, bc):
    return pl.pallas_call(
        _final_body,
        grid=(_T * _NB,),
        in_specs=[
            pl.BlockSpec((_BN, _F), lambda m: (m, 0)),
            pl.BlockSpec((_BN, _DW), lambda m: (m % _NB, 0)),
            pl.BlockSpec((_F, 2 * _FO), lambda m: (0, 0)),
            pl.BlockSpec((1, 2 * _FO), lambda m: (0, 0)),
        ],
        out_specs=pl.BlockSpec((_BN, _FO), lambda m: (m, 0)),
        out_shape=jax.ShapeDtypeStruct((_T * _N, _FO), jnp.float32),
    )(S, deg_w, Wc, bc)


def kernel(X, edge_index, W_gate, b_gate, W_upd, b_upd):
    dst = edge_index[1].reshape(_NSUB, _NCHUNK, _CH)
    comb = jnp.stack([edge_index[0].reshape(_NSUB, _ANCH, _ACH),
                      edge_index[1].reshape(_NSUB, _ANCH, _ACH)], axis=2)
    zeros = jnp.zeros((_N, _DW), jnp.float32)
    ones = jnp.ones((_CH, _DW), jnp.float32)
    deg_w = _deg_call(dst, zeros, ones)
    Xn = _prescale(X, deg_w)
    S = _agg_call(Xn, comb)
    Wc = jnp.concatenate([W_gate[:_F, _FO:], W_upd[:_F]], axis=1)
    bc = jnp.concatenate([b_gate[_FO:], b_upd]).reshape(1, 2 * _FO)
    H = _final(S, deg_w, Wc, bc)
    return H.reshape(_T, _N, _FO)
